# Initial kernel scaffold; baseline (speedup 1.0000x reference)
import jax
import jax.numpy as jnp
from jax.experimental import pallas as pl


def kernel(x, xyz):
    raise NotImplementedError("write your pallas kernel here")


# speedup vs baseline: 1.0021x; 1.0021x over previous
import jax
import jax.numpy as jnp
from jax import lax
from jax.experimental import pallas as pl

NUM_GROUP = 512
GROUP_SIZE = 32


def _index_points(points, idx):
    return jax.vmap(lambda p, i: p[i])(points, idx)


def _square_distance(src, dst):
    return (jnp.sum(src * src, -1)[..., None]
            + jnp.sum(dst * dst, -1)[:, None, :]
            - 2.0 * jnp.einsum('bmc,bnc->bmn', src, dst))


def _fps(xyz, npoint):
    b, n, _ = xyz.shape

    def step(carry, _):
        dist, farthest = carry
        centroid = xyz[jnp.arange(b), farthest]
        d = jnp.sum((xyz - centroid[:, None, :]) ** 2, -1)
        dist = jnp.minimum(dist, d)
        new_far = jnp.argmax(dist, axis=-1).astype(jnp.int32)
        return (dist, new_far), farthest

    init = (jnp.full((b, n), 1e10, dtype=xyz.dtype), jnp.zeros((b,), dtype=jnp.int32))
    _, idxs = lax.scan(step, init, None, length=npoint)
    return jnp.transpose(idxs)


def _part1by2(v):
    v = v & jnp.uint32(0x3FF)
    v = (v | (v << 16)) & jnp.uint32(0x030000FF)
    v = (v | (v << 8)) & jnp.uint32(0x0300F00F)
    v = (v | (v << 4)) & jnp.uint32(0x030C30C3)
    v = (v | (v << 2)) & jnp.uint32(0x09249249)
    return v


def _morton_argsort(points):
    mn = points.min(axis=1, keepdims=True)
    mx = points.max(axis=1, keepdims=True)
    scaled = (points - mn) / jnp.maximum(mx - mn, 1e-9)
    q = jnp.clip(scaled * 1023, 0, 1023).astype(jnp.uint32)
    code = (_part1by2(q[..., 0]) << 2) | (_part1by2(q[..., 1]) << 1) | _part1by2(q[..., 2])
    return jnp.argsort(code, axis=1)


def _sq_body(xyz_ref, out_ref):
    v = xyz_ref[...]
    out_ref[...] = jnp.sum(v * v, axis=1)


def _sq_norms(pts):
    b, n, c = pts.shape
    return pl.pallas_call(
        _sq_body,
        out_shape=jax.ShapeDtypeStruct((b, n), pts.dtype),
    )(jnp.swapaxes(pts, 1, 2))


def kernel(x, xyz):
    b, n, c = xyz.shape
    fps_idx = _fps(xyz, NUM_GROUP)
    center = _index_points(xyz, fps_idx)
    new_points = _index_points(x, fps_idx)
    dist = (_sq_norms(center)[..., None] + _sq_norms(xyz)[:, None, :]
            - 2.0 * jnp.einsum('bmc,bnc->bmn', center, xyz))
    _, idx = lax.top_k(-dist, GROUP_SIZE)
    idx = idx + jnp.arange(b).reshape(-1, 1, 1) * n
    idx = idx.reshape(-1)
    neighborhood = x.reshape(b * n, -1)[idx, :]
    neighborhood = neighborhood.reshape(b, NUM_GROUP, GROUP_SIZE, c)

    sorted_indices = _morton_argsort(center)
    sorted_flat = (sorted_indices + jnp.arange(b)[:, None] * NUM_GROUP).reshape(-1)
    neighborhood = neighborhood.reshape(b * NUM_GROUP, GROUP_SIZE, c)[sorted_flat]
    neighborhood = neighborhood.reshape(b, NUM_GROUP, GROUP_SIZE, c)
    ctr_s = new_points.reshape(b * NUM_GROUP, c)[sorted_flat].reshape(b, NUM_GROUP, c)
    center_out = center.reshape(b * NUM_GROUP, c)[sorted_flat].reshape(b, NUM_GROUP, c)

    out = jnp.concatenate(
        [neighborhood - ctr_s[:, :, None, :],
         jnp.broadcast_to(ctr_s[:, :, None, :], neighborhood.shape)], axis=-1)
    return out, center_out


# speedup vs baseline: 1.8234x; 1.8195x over previous
import functools

import jax
import jax.numpy as jnp
from jax import lax
from jax.experimental import pallas as pl
from jax.experimental.pallas import tpu as pltpu

NUM_GROUP = 512
GROUP_SIZE = 32
B, N, C = 8, 8192, 3


def _fps_body(xyz_ref, x_ref, center_ref, npts_ref, dist_ref):
    lanes = lax.broadcasted_iota(jnp.int32, (B, N), 1)
    dist_ref[...] = jnp.full((B, N), 1e10, jnp.float32)

    def step(g, far):
        onehot = lanes == far
        cs = []
        for ci in range(3):
            cc = jnp.sum(jnp.where(onehot, xyz_ref[ci], 0.0), axis=1, keepdims=True)
            cs.append(cc)
            center_ref[ci, pl.ds(g, 1), :] = cc.T
            fc = jnp.sum(jnp.where(onehot, x_ref[ci], 0.0), axis=1, keepdims=True)
            npts_ref[ci, pl.ds(g, 1), :] = fc.T
        d0 = xyz_ref[0] - cs[0]
        d1 = xyz_ref[1] - cs[1]
        d2 = xyz_ref[2] - cs[2]
        d = d0 * d0 + d1 * d1 + d2 * d2
        nd = jnp.minimum(dist_ref[...], d)
        dist_ref[...] = nd
        m = jnp.max(nd, axis=1, keepdims=True)
        far_new = jnp.min(jnp.where(nd == m, lanes, N), axis=1, keepdims=True)
        return far_new

    lax.fori_loop(0, NUM_GROUP, step, jnp.zeros((B, 1), jnp.int32))


def _fps_pallas(x, xyz):
    xyz_t = jnp.transpose(xyz, (2, 0, 1))
    x_t = jnp.transpose(x, (2, 0, 1))
    center_t, npts_t = pl.pallas_call(
        _fps_body,
        out_shape=[jax.ShapeDtypeStruct((C, NUM_GROUP, B), jnp.float32),
                   jax.ShapeDtypeStruct((C, NUM_GROUP, B), jnp.float32)],
        scratch_shapes=[pltpu.VMEM((B, N), jnp.float32)],
    )(xyz_t, x_t)
    center = jnp.transpose(center_t, (2, 1, 0))
    npts = jnp.transpose(npts_t, (2, 1, 0))
    return center, npts


def _part1by2(v):
    v = v & jnp.uint32(0x3FF)
    v = (v | (v << 16)) & jnp.uint32(0x030000FF)
    v = (v | (v << 8)) & jnp.uint32(0x0300F00F)
    v = (v | (v << 4)) & jnp.uint32(0x030C30C3)
    v = (v | (v << 2)) & jnp.uint32(0x09249249)
    return v


def _morton_argsort(points):
    mn = points.min(axis=1, keepdims=True)
    mx = points.max(axis=1, keepdims=True)
    scaled = (points - mn) / jnp.maximum(mx - mn, 1e-9)
    q = jnp.clip(scaled * 1023, 0, 1023).astype(jnp.uint32)
    code = (_part1by2(q[..., 0]) << 2) | (_part1by2(q[..., 1]) << 1) | _part1by2(q[..., 2])
    return jnp.argsort(code, axis=1)


def kernel(x, xyz):
    b, n, c = xyz.shape
    center, new_points = _fps_pallas(x, xyz)
    dist = (jnp.sum(center * center, -1)[..., None]
            + jnp.sum(xyz * xyz, -1)[:, None, :]
            - 2.0 * jnp.einsum('bmc,bnc->bmn', center, xyz))
    _, idx = lax.top_k(-dist, GROUP_SIZE)
    idx = idx + jnp.arange(b).reshape(-1, 1, 1) * n
    idx = idx.reshape(-1)
    neighborhood = x.reshape(b * n, -1)[idx, :]
    neighborhood = neighborhood.reshape(b, NUM_GROUP, GROUP_SIZE, c)

    sorted_indices = _morton_argsort(center)
    sorted_flat = (sorted_indices + jnp.arange(b)[:, None] * NUM_GROUP).reshape(-1)
    neighborhood = neighborhood.reshape(b * NUM_GROUP, GROUP_SIZE, c)[sorted_flat]
    neighborhood = neighborhood.reshape(b, NUM_GROUP, GROUP_SIZE, c)
    ctr_s = new_points.reshape(b * NUM_GROUP, c)[sorted_flat].reshape(b, NUM_GROUP, c)
    center_out = center.reshape(b * NUM_GROUP, c)[sorted_flat].reshape(b, NUM_GROUP, c)

    out = jnp.concatenate(
        [neighborhood - ctr_s[:, :, None, :],
         jnp.broadcast_to(ctr_s[:, :, None, :], neighborhood.shape)], axis=-1)
    return out, center_out
